# baseline (speedup 1.0000x reference)
import functools

import jax
import jax.numpy as jnp
import numpy as np
from jax import lax
from jax.experimental import pallas as pl
from jax.experimental.pallas import tpu as pltpu
from jax.experimental.pallas import tpu_sc as plsc

N = 10000
E = 320000
G = 64
H = 128
HEADS = 4
DH = H // HEADS

ACC_W = 144
CHUNK = 128
NW = 32
BE = 2000
BN_ = 2000
GE = E // BE
GN = N // BN_
TILES = 16
NPAD = 10240
RPT = NPAD // TILES
ZR = 128


def _gelu(x):
    return 0.5 * x * (1.0 + lax.erf(x * np.float32(0.7071067811865476)))


def _mmb(a, b):
    return lax.dot_general(
        a.astype(jnp.bfloat16), b.astype(jnp.bfloat16),
        ((( 1,), (0,)), ((), ())),
        preferred_element_type=jnp.float32)


def _sc_gather(table, idx, rows, width):
    nch = rows // CHUNK
    trips = (nch + NW - 1) // NW
    mesh = plsc.VectorSubcoreMesh(core_axis_name="c", subcore_axis_name="s")

    @functools.partial(
        pl.kernel,
        mesh=mesh,
        out_type=jax.ShapeDtypeStruct((rows, width), jnp.float32),
        scratch_types=[
            pltpu.VMEM((CHUNK,), jnp.int32),
            pltpu.VMEM((CHUNK, width), jnp.float32),
            pltpu.SemaphoreType.DMA,
        ],
    )
    def k(table_hbm, idx_hbm, out_hbm, idx_v, rows_v, sem):
        wid = lax.axis_index("s") * 2 + lax.axis_index("c")

        def body(t, carry):
            ch = wid + t * NW

            @pl.when(ch < nch)
            def _():
                base = ch * CHUNK
                pltpu.sync_copy(idx_hbm.at[pl.ds(base, CHUNK)], idx_v)
                pltpu.async_copy(table_hbm.at[idx_v], rows_v, sem).wait()
                pltpu.sync_copy(rows_v, out_hbm.at[pl.ds(base, CHUNK)])

            return carry

        lax.fori_loop(0, trips, body, 0)

    return k(table, idx)


def _sc_scatter128(vals, idx):
    nch = E // CHUNK
    trips = (nch + NW - 1) // NW
    mesh = plsc.VectorSubcoreMesh(core_axis_name="c", subcore_axis_name="s")

    @functools.partial(
        pl.kernel,
        mesh=mesh,
        out_type=jax.ShapeDtypeStruct((2, NPAD, H), jnp.float32),
        scratch_types=[
            pltpu.VMEM((CHUNK,), jnp.int32),
            pltpu.VMEM((CHUNK, H), jnp.float32),
            pltpu.VMEM((ZR, H), jnp.float32),
            pltpu.VMEM_SHARED((NPAD, H), jnp.float32),
            pltpu.SemaphoreType.DMA,
        ],
    )
    def k(vals_hbm, idx_hbm, out_hbm, idx_v, val_v, zbuf, accum, sem):
        c = lax.axis_index("c")
        s = lax.axis_index("s")
        wid = s * 2 + c

        def zrow(i, carry):
            def zcol(j, cc):
                zbuf[i, pl.ds(j * 16, 16)] = jnp.zeros((16,), jnp.float32)
                return cc

            lax.fori_loop(0, H // 16, zcol, 0)
            return carry

        lax.fori_loop(0, ZR, zrow, 0)
        row0 = s * RPT
        for r in range(RPT // ZR):
            pltpu.sync_copy(zbuf, accum.at[pl.ds(row0 + r * ZR, ZR)])
        plsc.subcore_barrier()

        def body(t, carry):
            ch = wid + t * NW

            @pl.when(ch < nch)
            def _():
                base = ch * CHUNK
                pltpu.sync_copy(idx_hbm.at[pl.ds(base, CHUNK)], idx_v)
                pltpu.sync_copy(vals_hbm.at[pl.ds(base, CHUNK)], val_v)
                pltpu.sync_copy(val_v, accum.at[idx_v], add=True)

            return carry

        lax.fori_loop(0, trips, body, 0)
        plsc.subcore_barrier()
        pltpu.sync_copy(accum.at[pl.ds(row0, RPT)],
                        out_hbm.at[c, pl.ds(row0, RPT)])

    return k(vals, idx)


def _e1_body(hE, hvs, hvd, w1, b1, w2, b2, w3, b3,
             v1, vb1, v2, vb2, v3, vb3, rep8, out, ew_out):
    he = hE[...]
    xs = hvs[...]
    xd = hvd[...]
    xw = jnp.concatenate([xs, he, xd], axis=1)
    a = _gelu(_mmb(xw, w1[...]) + b1[...])
    a = _gelu(_mmb(a, w2[...]) + b2[...])
    w8 = _mmb(a, w3[...]) + b3[...]
    xv = jnp.concatenate([he, xd], axis=1)
    v = _gelu(_mmb(xv, v1[...]) + vb1[...])
    v = _gelu(_mmb(v, v2[...]) + vb2[...])
    v = _mmb(v, v3[...]) + vb3[...]
    mask = (lax.broadcasted_iota(jnp.int32, (1, 8), 1) < HEADS)
    ew8 = jnp.exp(w8) * mask.astype(jnp.float32)
    out[...] = v * (ew8 @ rep8[...])
    ew_out[...] = jnp.concatenate(
        [ew8, jnp.zeros((ew8.shape[0], H - 8), jnp.float32)], axis=1)


def _a1_body(part, spart, hv, wo, rep4, t0_out, sums):
    i = pl.program_id(0)
    p = part[...]
    u = p[0] + p[1]
    q = spart[...]
    s4 = q[0, :, :HEADS] + q[1, :, :HEADS]
    agg = u / (s4 @ rep4[...] + 1e-9)
    t0 = hv[...] + agg @ wo[...]
    t0_out[...] = t0

    @pl.when(i == 0)
    def _():
        sums[...] = jnp.zeros_like(sums)

    sums[...] += jnp.concatenate(
        [jnp.sum(t0, axis=0, keepdims=True),
         jnp.sum(t0 * t0, axis=0, keepdims=True)], axis=0)


def _affine(sums, n, g, b):
    m = sums[0:1] / n
    v = sums[1:2] / n - m * m
    a = g / jnp.sqrt(v + 1e-5)
    return a, b - m * a


def _a2_body(t0, sums0, g0, b0, d0w, d0b, d1w, d1b, t2_out, sums):
    i = pl.program_id(0)
    a, bsh = _affine(sums0[...], float(N), g0[...], b0[...])
    t1 = a * t0[...] + bsh
    h = _gelu(t1 @ d0w[...] + d0b[...])
    t2 = t1 + h @ d1w[...] + d1b[...]
    t2_out[...] = t2

    @pl.when(i == 0)
    def _():
        sums[...] = jnp.zeros_like(sums)

    sums[...] += jnp.concatenate(
        [jnp.sum(t2, axis=0, keepdims=True),
         jnp.sum(t2 * t2, axis=0, keepdims=True)], axis=0)


def _b_body(t2, sums1, g1, b1, bid,
            vw1, vb1, vw2, vb2, vw3, vb3,
            ew1, eb1, ew2, eb2, ew3, eb3,
            hv2_out, hv3_out, tab_out):
    a, bsh = _affine(sums1[...], float(N), g1[...], b1[...])
    hv2 = a * t2[...] + bsh
    onehot = jnp.equal(
        bid[...], lax.broadcasted_iota(jnp.int32, (N, G), 1)
    ).astype(jnp.float32)
    dn = (((0,), (0,)), ((), ()))
    csum = lax.dot_general(onehot, hv2, dn)
    cnt = lax.dot_general(onehot, jnp.ones((N, 1), jnp.float32), dn)
    c = csum / jnp.maximum(cnt, 1.0)

    def mlp3(x, w1, bb1, w2, bb2, w3, bb3):
        x = _gelu(x @ w1[...] + bb1[...])
        x = _gelu(x @ w2[...] + bb2[...])
        return x @ w3[...] + bb3[...]

    svg = jax.nn.sigmoid(mlp3(c, vw1, vb1, vw2, vb2, vw3, vb3))
    c2 = c * svg
    seg = jax.nn.sigmoid(mlp3(c2, ew1, eb1, ew2, eb2, ew3, eb3))
    hv3 = hv2 * (onehot @ svg)
    hv2_out[...] = hv2
    hv3_out[...] = hv3
    tab_out[...] = jnp.concatenate([hv2, onehot @ seg], axis=1)


def _e2_body(hE, g2s, gd, w1, b1, w2, b2, w3, b3, x_out, sums):
    i = pl.program_id(0)
    he = hE[...]
    xin = jnp.concatenate([g2s[...], he, gd[...]], axis=1)
    a = _gelu(_mmb(xin, w1[...]) + b1[...])
    a = _gelu(_mmb(a, w2[...]) + b2[...])
    x = he + (_mmb(a, w3[...]) + b3[...])
    x_out[...] = x

    @pl.when(i == 0)
    def _():
        sums[...] = jnp.zeros_like(sums)

    sums[...] += jnp.concatenate(
        [jnp.sum(x, axis=0, keepdims=True),
         jnp.sum(x * x, axis=0, keepdims=True)], axis=0)


def _f_body(x, gate, sumse, ge, be, out):
    a, bsh = _affine(sumse[...], float(E), ge[...], be[...])
    out[...] = (a * x[...] + bsh) * gate[...]


def _spec(shape, imap):
    return pl.BlockSpec(shape, imap)


def _row2(x):
    return x.reshape(1, -1)


def kernel(h_V, h_E, edge_idx, batch_id, params):
    src = edge_idx[0]
    dst = edge_idx[1]
    idx_all = edge_idx.reshape(2 * E)

    rep8 = np.zeros((8, H), np.float32)
    rep4 = np.zeros((HEADS, H), np.float32)
    for hh in range(HEADS):
        rep8[hh, hh * DH:(hh + 1) * DH] = 1.0
        rep4[hh, hh * DH:(hh + 1) * DH] = 1.0
    rep8 = jnp.asarray(rep8)
    rep4 = jnp.asarray(rep4)

    pb = params["att_Bias"]
    w3p = jnp.pad(pb[2]["w"], ((0, 0), (0, 8 - HEADS)))
    b3p = _row2(jnp.pad(pb[2]["b"], (0, 8 - HEADS)))
    pv = params["att_WV"]
    pe = params["edge"]
    d0, d1 = params["dense"]

    gath = _sc_gather(h_V, idx_all, 2 * E, H)
    hvs, hvd = gath[:E], gath[E:]

    wspec = lambda shp: _spec(shp, lambda i: (0, 0))
    p_acc, ew_acc = pl.pallas_call(
        _e1_body,
        grid=(GE,),
        in_specs=[
            _spec((BE, H), lambda i: (i, 0)),
            _spec((BE, H), lambda i: (i, 0)),
            _spec((BE, H), lambda i: (i, 0)),
            wspec((3 * H, H)), wspec((1, H)),
            wspec((H, H)), wspec((1, H)),
            wspec((H, 8)), wspec((1, 8)),
            wspec((2 * H, H)), wspec((1, H)),
            wspec((H, H)), wspec((1, H)),
            wspec((H, H)), wspec((1, H)),
            wspec((8, H)),
        ],
        out_specs=[_spec((BE, H), lambda i: (i, 0)),
                   _spec((BE, H), lambda i: (i, 0))],
        out_shape=[jax.ShapeDtypeStruct((E, H), jnp.float32),
                   jax.ShapeDtypeStruct((E, H), jnp.float32)],
    )(h_E, hvs, hvd,
      pb[0]["w"], _row2(pb[0]["b"]), pb[1]["w"], _row2(pb[1]["b"]), w3p, b3p,
      pv[0]["w"], _row2(pv[0]["b"]), pv[1]["w"], _row2(pv[1]["b"]),
      pv[2]["w"], _row2(pv[2]["b"]), rep8)

    partials = _sc_scatter128(p_acc, src)
    spartials = _sc_scatter128(ew_acc, src)

    t0, sums0 = pl.pallas_call(
        _a1_body,
        grid=(GN,),
        in_specs=[
            _spec((2, BN_, H), lambda i: (0, i, 0)),
            _spec((2, BN_, H), lambda i: (0, i, 0)),
            _spec((BN_, H), lambda i: (i, 0)),
            wspec((H, H)),
            wspec((HEADS, H)),
        ],
        out_specs=[_spec((BN_, H), lambda i: (i, 0)),
                   _spec((2, H), lambda i: (0, 0))],
        out_shape=[jax.ShapeDtypeStruct((N, H), jnp.float32),
                   jax.ShapeDtypeStruct((2, H), jnp.float32)],
    )(partials, spartials, h_V, params["att_WO"], rep4)

    t2raw, sums1 = pl.pallas_call(
        _a2_body,
        grid=(GN,),
        in_specs=[
            _spec((BN_, H), lambda i: (i, 0)),
            wspec((2, H)),
            wspec((1, H)), wspec((1, H)),
            wspec((H, 4 * H)), wspec((1, 4 * H)),
            wspec((4 * H, H)), wspec((1, H)),
        ],
        out_specs=[_spec((BN_, H), lambda i: (i, 0)),
                   _spec((2, H), lambda i: (0, 0))],
        out_shape=[jax.ShapeDtypeStruct((N, H), jnp.float32),
                   jax.ShapeDtypeStruct((2, H), jnp.float32)],
    )(t0, sums0, _row2(params["bn0_g"]), _row2(params["bn0_b"]),
      d0["w"], _row2(d0["b"]), d1["w"], _row2(d1["b"]))

    pvg = params["vgate"]
    peg = params["egate"]
    hv2, hv3, tab = pl.pallas_call(
        _b_body,
        grid=(1,),
        in_specs=[
            _spec((N, H), lambda i: (0, 0)),
            wspec((2, H)),
            wspec((1, H)), wspec((1, H)),
            _spec((N, 1), lambda i: (0, 0)),
            wspec((H, H)), wspec((1, H)),
            wspec((H, H)), wspec((1, H)),
            wspec((H, H)), wspec((1, H)),
            wspec((H, H)), wspec((1, H)),
            wspec((H, H)), wspec((1, H)),
            wspec((H, H)), wspec((1, H)),
        ],
        out_specs=[_spec((N, H), lambda i: (0, 0)),
                   _spec((N, H), lambda i: (0, 0)),
                   _spec((N, 2 * H), lambda i: (0, 0))],
        out_shape=[jax.ShapeDtypeStruct((N, H), jnp.float32),
                   jax.ShapeDtypeStruct((N, H), jnp.float32),
                   jax.ShapeDtypeStruct((N, 2 * H), jnp.float32)],
    )(t2raw, sums1, _row2(params["bn1_g"]), _row2(params["bn1_b"]),
      batch_id.reshape(N, 1),
      pvg[0]["w"], _row2(pvg[0]["b"]), pvg[1]["w"], _row2(pvg[1]["b"]),
      pvg[2]["w"], _row2(pvg[2]["b"]),
      peg[0]["w"], _row2(peg[0]["b"]), peg[1]["w"], _row2(peg[1]["b"]),
      peg[2]["w"], _row2(peg[2]["b"]))

    g2 = _sc_gather(tab, src, E, 2 * H)
    gd = _sc_gather(hv2, dst, E, H)

    x, sumse = pl.pallas_call(
        _e2_body,
        grid=(GE,),
        in_specs=[
            _spec((BE, H), lambda i: (i, 0)),
            _spec((BE, H), lambda i: (i, 0)),
            _spec((BE, H), lambda i: (i, 0)),
            wspec((3 * H, H)), wspec((1, H)),
            wspec((H, H)), wspec((1, H)),
            wspec((H, H)), wspec((1, H)),
        ],
        out_specs=[_spec((BE, H), lambda i: (i, 0)),
                   _spec((2, H), lambda i: (0, 0))],
        out_shape=[jax.ShapeDtypeStruct((E, H), jnp.float32),
                   jax.ShapeDtypeStruct((2, H), jnp.float32)],
    )(h_E, g2, gd,
      pe[0]["w"], _row2(pe[0]["b"]), pe[1]["w"], _row2(pe[1]["b"]),
      pe[2]["w"], _row2(pe[2]["b"]))

    h_E_out = pl.pallas_call(
        _f_body,
        grid=(GE,),
        in_specs=[
            _spec((BE, H), lambda i: (i, 0)),
            _spec((BE, H), lambda i: (i, 1)),
            wspec((2, H)),
            wspec((1, H)), wspec((1, H)),
        ],
        out_specs=_spec((BE, H), lambda i: (i, 0)),
        out_shape=jax.ShapeDtypeStruct((E, H), jnp.float32),
    )(x, g2, sumse, _row2(params["ebn_g"]), _row2(params["ebn_b"]))

    return (hv3, h_E_out)


# speedup vs baseline: 1.3784x; 1.3784x over previous
import functools

import jax
import jax.numpy as jnp
import numpy as np
from jax import lax
from jax.experimental import pallas as pl
from jax.experimental.pallas import tpu as pltpu
from jax.experimental.pallas import tpu_sc as plsc

N = 10000
E = 320000
G = 64
H = 128
HEADS = 4
DH = H // HEADS

ACC_W = 144
CHUNK = 128
NW = 32
BE = 2000
BN_ = 2000
GE = E // BE
GN = N // BN_
TILES = 16
NPAD = 10240
RPT = NPAD // TILES
ZR = 128


def _gelu(x):
    return 0.5 * x * (1.0 + lax.erf(x * np.float32(0.7071067811865476)))


def _mmb(a, b):
    return lax.dot_general(
        a.astype(jnp.bfloat16), b.astype(jnp.bfloat16),
        ((( 1,), (0,)), ((), ())),
        preferred_element_type=jnp.float32)


def _sc_gather(table, idx, rows, width, kc):
    R = kc * CHUNK
    U = rows // R
    pairs = (U + 2 * NW - 1) // (2 * NW)
    mesh = plsc.VectorSubcoreMesh(core_axis_name="c", subcore_axis_name="s")

    @functools.partial(
        pl.kernel,
        mesh=mesh,
        out_type=jax.ShapeDtypeStruct((rows, width), jnp.float32),
        scratch_types=[
            pltpu.VMEM((R,), jnp.int32),
            pltpu.VMEM((R,), jnp.int32),
            pltpu.VMEM((R, width), jnp.float32),
            pltpu.VMEM((R, width), jnp.float32),
            pltpu.SemaphoreType.DMA,
            pltpu.SemaphoreType.DMA,
            pltpu.SemaphoreType.DMA,
            pltpu.SemaphoreType.DMA,
            pltpu.SemaphoreType.DMA,
            pltpu.SemaphoreType.DMA,
        ],
    )
    def k(table_hbm, idx_hbm, out_hbm, i0, i1, b0, b1,
          is0, is1, gs0, gs1, os0, os1):
        wid = lax.axis_index("s") * 2 + lax.axis_index("c")
        ibufs = (i0, i1)
        bufs = (b0, b1)
        isems = (is0, is1)
        gsems = (gs0, gs1)
        osems = (os0, os1)

        def body(t, carry):
            for b in range(2):
                u = wid + (2 * t + b) * NW

                @pl.when(u < U)
                def _(u=u, b=b):
                    pltpu.async_copy(idx_hbm.at[pl.ds(u * R, R)],
                                     ibufs[b], isems[b])

            for b in range(2):
                u = wid + (2 * t + b) * NW

                @pl.when(u < U)
                def _(u=u, b=b):
                    @pl.when(u >= 2 * NW)
                    def _():
                        pltpu.make_async_copy(
                            bufs[b], out_hbm.at[pl.ds((u - 2 * NW) * R, R)],
                            osems[b]).wait()

                    pltpu.make_async_copy(idx_hbm.at[pl.ds(u * R, R)],
                                          ibufs[b], isems[b]).wait()
                    for kk in range(kc):
                        pltpu.async_copy(
                            table_hbm.at[ibufs[b].at[pl.ds(kk * CHUNK, CHUNK)]],
                            bufs[b].at[pl.ds(kk * CHUNK, CHUNK)], gsems[b])

            for b in range(2):
                u = wid + (2 * t + b) * NW

                @pl.when(u < U)
                def _(u=u, b=b):
                    for kk in range(kc):
                        pltpu.make_async_copy(
                            table_hbm.at[ibufs[b].at[pl.ds(kk * CHUNK, CHUNK)]],
                            bufs[b].at[pl.ds(kk * CHUNK, CHUNK)],
                            gsems[b]).wait()
                    pltpu.async_copy(bufs[b], out_hbm.at[pl.ds(u * R, R)],
                                     osems[b])

            return carry

        lax.fori_loop(0, pairs, body, 0)

        jm = (U - 1 - wid) // NW
        for b in range(2):
            jb = jnp.where((jm % 2) == b, jm, jm - 1)

            @pl.when(jb >= 0)
            def _(b=b, jb=jb):
                ul = wid + jb * NW
                pltpu.make_async_copy(
                    bufs[b], out_hbm.at[pl.ds(ul * R, R)], osems[b]).wait()

    return k(table, idx)


def _sc_scatter128(vals, idx):
    nch = E // CHUNK
    trips = (nch + NW - 1) // NW
    mesh = plsc.VectorSubcoreMesh(core_axis_name="c", subcore_axis_name="s")

    @functools.partial(
        pl.kernel,
        mesh=mesh,
        out_type=jax.ShapeDtypeStruct((2, NPAD, H), jnp.float32),
        scratch_types=[
            pltpu.VMEM((CHUNK,), jnp.int32),
            pltpu.VMEM((CHUNK, H), jnp.float32),
            pltpu.VMEM((ZR, H), jnp.float32),
            pltpu.VMEM_SHARED((NPAD, H), jnp.float32),
            pltpu.SemaphoreType.DMA,
        ],
    )
    def k(vals_hbm, idx_hbm, out_hbm, idx_v, val_v, zbuf, accum, sem):
        c = lax.axis_index("c")
        s = lax.axis_index("s")
        wid = s * 2 + c

        def zrow(i, carry):
            def zcol(j, cc):
                zbuf[i, pl.ds(j * 16, 16)] = jnp.zeros((16,), jnp.float32)
                return cc

            lax.fori_loop(0, H // 16, zcol, 0)
            return carry

        lax.fori_loop(0, ZR, zrow, 0)
        row0 = s * RPT
        for r in range(RPT // ZR):
            pltpu.sync_copy(zbuf, accum.at[pl.ds(row0 + r * ZR, ZR)])
        plsc.subcore_barrier()

        def body(t, carry):
            ch = wid + t * NW

            @pl.when(ch < nch)
            def _():
                base = ch * CHUNK
                pltpu.sync_copy(idx_hbm.at[pl.ds(base, CHUNK)], idx_v)
                pltpu.sync_copy(vals_hbm.at[pl.ds(base, CHUNK)], val_v)
                pltpu.sync_copy(val_v, accum.at[idx_v], add=True)

            return carry

        lax.fori_loop(0, trips, body, 0)
        plsc.subcore_barrier()
        pltpu.sync_copy(accum.at[pl.ds(row0, RPT)],
                        out_hbm.at[c, pl.ds(row0, RPT)])

    return k(vals, idx)


def _e1_body(hE, hvs, hvd, w1, b1, w2, b2, w3, b3,
             v1, vb1, v2, vb2, v3, vb3, rep8, out, ew_out):
    he = hE[...]
    xs = hvs[...]
    xd = hvd[...]
    xw = jnp.concatenate([xs, he, xd], axis=1)
    a = _gelu(xw @ w1[...] + b1[...])
    a = _gelu(a @ w2[...] + b2[...])
    w8 = a @ w3[...] + b3[...]
    xv = jnp.concatenate([he, xd], axis=1)
    v = _gelu(xv @ v1[...] + vb1[...])
    v = _gelu(v @ v2[...] + vb2[...])
    v = v @ v3[...] + vb3[...]
    mask = (lax.broadcasted_iota(jnp.int32, (1, 8), 1) < HEADS)
    ew8 = jnp.exp(w8) * mask.astype(jnp.float32)
    out[...] = v * (ew8 @ rep8[...])
    ew_out[...] = jnp.concatenate(
        [ew8, jnp.zeros((ew8.shape[0], H - 8), jnp.float32)], axis=1)


def _a1_body(part, spart, hv, wo, rep4, t0_out, sums):
    i = pl.program_id(0)
    p = part[...]
    u = p[0] + p[1]
    q = spart[...]
    s4 = q[0, :, :HEADS] + q[1, :, :HEADS]
    agg = u / (s4 @ rep4[...] + 1e-9)
    t0 = hv[...] + agg @ wo[...]
    t0_out[...] = t0

    @pl.when(i == 0)
    def _():
        sums[...] = jnp.zeros_like(sums)

    sums[...] += jnp.concatenate(
        [jnp.sum(t0, axis=0, keepdims=True),
         jnp.sum(t0 * t0, axis=0, keepdims=True)], axis=0)


def _affine(sums, n, g, b):
    m = sums[0:1] / n
    v = sums[1:2] / n - m * m
    a = g / jnp.sqrt(v + 1e-5)
    return a, b - m * a


def _a2_body(t0, sums0, g0, b0, d0w, d0b, d1w, d1b, t2_out, sums):
    i = pl.program_id(0)
    a, bsh = _affine(sums0[...], float(N), g0[...], b0[...])
    t1 = a * t0[...] + bsh
    h = _gelu(t1 @ d0w[...] + d0b[...])
    t2 = t1 + h @ d1w[...] + d1b[...]
    t2_out[...] = t2

    @pl.when(i == 0)
    def _():
        sums[...] = jnp.zeros_like(sums)

    sums[...] += jnp.concatenate(
        [jnp.sum(t2, axis=0, keepdims=True),
         jnp.sum(t2 * t2, axis=0, keepdims=True)], axis=0)


def _b_body(t2, sums1, g1, b1, bid,
            vw1, vb1, vw2, vb2, vw3, vb3,
            ew1, eb1, ew2, eb2, ew3, eb3,
            hv2_out, hv3_out, tab_out):
    a, bsh = _affine(sums1[...], float(N), g1[...], b1[...])
    hv2 = a * t2[...] + bsh
    onehot = jnp.equal(
        bid[...], lax.broadcasted_iota(jnp.int32, (N, G), 1)
    ).astype(jnp.float32)
    dn = (((0,), (0,)), ((), ()))
    csum = lax.dot_general(onehot, hv2, dn)
    cnt = lax.dot_general(onehot, jnp.ones((N, 1), jnp.float32), dn)
    c = csum / jnp.maximum(cnt, 1.0)

    def mlp3(x, w1, bb1, w2, bb2, w3, bb3):
        x = _gelu(x @ w1[...] + bb1[...])
        x = _gelu(x @ w2[...] + bb2[...])
        return x @ w3[...] + bb3[...]

    svg = jax.nn.sigmoid(mlp3(c, vw1, vb1, vw2, vb2, vw3, vb3))
    c2 = c * svg
    seg = jax.nn.sigmoid(mlp3(c2, ew1, eb1, ew2, eb2, ew3, eb3))
    hv3 = hv2 * (onehot @ svg)
    hv2_out[...] = hv2
    hv3_out[...] = hv3
    tab_out[...] = jnp.concatenate([hv2, onehot @ seg], axis=1)


def _e2_body(hE, g2s, gd, w1, b1, w2, b2, w3, b3, x_out, sums):
    i = pl.program_id(0)
    he = hE[...]
    xin = jnp.concatenate([g2s[...], he, gd[...]], axis=1)
    a = _gelu(xin @ w1[...] + b1[...])
    a = _gelu(a @ w2[...] + b2[...])
    x = he + (a @ w3[...] + b3[...])
    x_out[...] = x

    @pl.when(i == 0)
    def _():
        sums[...] = jnp.zeros_like(sums)

    sums[...] += jnp.concatenate(
        [jnp.sum(x, axis=0, keepdims=True),
         jnp.sum(x * x, axis=0, keepdims=True)], axis=0)


def _f_body(x, gate, sumse, ge, be, out):
    a, bsh = _affine(sumse[...], float(E), ge[...], be[...])
    out[...] = (a * x[...] + bsh) * gate[...]


def _spec(shape, imap):
    return pl.BlockSpec(shape, imap)


def _row2(x):
    return x.reshape(1, -1)


def kernel(h_V, h_E, edge_idx, batch_id, params):
    src = edge_idx[0]
    dst = edge_idx[1]
    idx_all = edge_idx.reshape(2 * E)

    rep8 = np.zeros((8, H), np.float32)
    rep4 = np.zeros((HEADS, H), np.float32)
    for hh in range(HEADS):
        rep8[hh, hh * DH:(hh + 1) * DH] = 1.0
        rep4[hh, hh * DH:(hh + 1) * DH] = 1.0
    rep8 = jnp.asarray(rep8)
    rep4 = jnp.asarray(rep4)

    pb = params["att_Bias"]
    w3p = jnp.pad(pb[2]["w"], ((0, 0), (0, 8 - HEADS)))
    b3p = _row2(jnp.pad(pb[2]["b"], (0, 8 - HEADS)))
    pv = params["att_WV"]
    pe = params["edge"]
    d0, d1 = params["dense"]

    R1 = 3 * CHUNK
    rows1 = ((2 * E + R1 - 1) // R1) * R1
    idx1 = jnp.pad(idx_all, (0, rows1 - 2 * E))
    gath = _sc_gather(h_V, idx1, rows1, H, 3)

    wspec = lambda shp: _spec(shp, lambda i: (0, 0))
    p_acc, ew_acc = pl.pallas_call(
        _e1_body,
        grid=(GE,),
        in_specs=[
            _spec((BE, H), lambda i: (i, 0)),
            _spec((BE, H), lambda i: (i, 0)),
            _spec((BE, H), lambda i: (i + E // BE, 0)),
            wspec((3 * H, H)), wspec((1, H)),
            wspec((H, H)), wspec((1, H)),
            wspec((H, 8)), wspec((1, 8)),
            wspec((2 * H, H)), wspec((1, H)),
            wspec((H, H)), wspec((1, H)),
            wspec((H, H)), wspec((1, H)),
            wspec((8, H)),
        ],
        out_specs=[_spec((BE, H), lambda i: (i, 0)),
                   _spec((BE, H), lambda i: (i, 0))],
        out_shape=[jax.ShapeDtypeStruct((E, H), jnp.float32),
                   jax.ShapeDtypeStruct((E, H), jnp.float32)],
    )(h_E, gath, gath,
      pb[0]["w"], _row2(pb[0]["b"]), pb[1]["w"], _row2(pb[1]["b"]), w3p, b3p,
      pv[0]["w"], _row2(pv[0]["b"]), pv[1]["w"], _row2(pv[1]["b"]),
      pv[2]["w"], _row2(pv[2]["b"]), rep8)

    partials = _sc_scatter128(p_acc, src)
    spartials = _sc_scatter128(ew_acc, src)

    t0, sums0 = pl.pallas_call(
        _a1_body,
        grid=(GN,),
        in_specs=[
            _spec((2, BN_, H), lambda i: (0, i, 0)),
            _spec((2, BN_, H), lambda i: (0, i, 0)),
            _spec((BN_, H), lambda i: (i, 0)),
            wspec((H, H)),
            wspec((HEADS, H)),
        ],
        out_specs=[_spec((BN_, H), lambda i: (i, 0)),
                   _spec((2, H), lambda i: (0, 0))],
        out_shape=[jax.ShapeDtypeStruct((N, H), jnp.float32),
                   jax.ShapeDtypeStruct((2, H), jnp.float32)],
    )(partials, spartials, h_V, params["att_WO"], rep4)

    t2raw, sums1 = pl.pallas_call(
        _a2_body,
        grid=(GN,),
        in_specs=[
            _spec((BN_, H), lambda i: (i, 0)),
            wspec((2, H)),
            wspec((1, H)), wspec((1, H)),
            wspec((H, 4 * H)), wspec((1, 4 * H)),
            wspec((4 * H, H)), wspec((1, H)),
        ],
        out_specs=[_spec((BN_, H), lambda i: (i, 0)),
                   _spec((2, H), lambda i: (0, 0))],
        out_shape=[jax.ShapeDtypeStruct((N, H), jnp.float32),
                   jax.ShapeDtypeStruct((2, H), jnp.float32)],
    )(t0, sums0, _row2(params["bn0_g"]), _row2(params["bn0_b"]),
      d0["w"], _row2(d0["b"]), d1["w"], _row2(d1["b"]))

    pvg = params["vgate"]
    peg = params["egate"]
    hv2, hv3, tab = pl.pallas_call(
        _b_body,
        grid=(1,),
        in_specs=[
            _spec((N, H), lambda i: (0, 0)),
            wspec((2, H)),
            wspec((1, H)), wspec((1, H)),
            _spec((N, 1), lambda i: (0, 0)),
            wspec((H, H)), wspec((1, H)),
            wspec((H, H)), wspec((1, H)),
            wspec((H, H)), wspec((1, H)),
            wspec((H, H)), wspec((1, H)),
            wspec((H, H)), wspec((1, H)),
            wspec((H, H)), wspec((1, H)),
        ],
        out_specs=[_spec((N, H), lambda i: (0, 0)),
                   _spec((N, H), lambda i: (0, 0)),
                   _spec((N, 2 * H), lambda i: (0, 0))],
        out_shape=[jax.ShapeDtypeStruct((N, H), jnp.float32),
                   jax.ShapeDtypeStruct((N, H), jnp.float32),
                   jax.ShapeDtypeStruct((N, 2 * H), jnp.float32)],
    )(t2raw, sums1, _row2(params["bn1_g"]), _row2(params["bn1_b"]),
      batch_id.reshape(N, 1),
      pvg[0]["w"], _row2(pvg[0]["b"]), pvg[1]["w"], _row2(pvg[1]["b"]),
      pvg[2]["w"], _row2(pvg[2]["b"]),
      peg[0]["w"], _row2(peg[0]["b"]), peg[1]["w"], _row2(peg[1]["b"]),
      peg[2]["w"], _row2(peg[2]["b"]))

    g2 = _sc_gather(tab, src, E, 2 * H, 1)
    rows_d = ((E + R1 - 1) // R1) * R1
    gd = _sc_gather(hv2, jnp.pad(dst, (0, rows_d - E)), rows_d, H, 3)

    x, sumse = pl.pallas_call(
        _e2_body,
        grid=(GE,),
        in_specs=[
            _spec((BE, H), lambda i: (i, 0)),
            _spec((BE, H), lambda i: (i, 0)),
            _spec((BE, H), lambda i: (i, 0)),
            wspec((3 * H, H)), wspec((1, H)),
            wspec((H, H)), wspec((1, H)),
            wspec((H, H)), wspec((1, H)),
        ],
        out_specs=[_spec((BE, H), lambda i: (i, 0)),
                   _spec((2, H), lambda i: (0, 0))],
        out_shape=[jax.ShapeDtypeStruct((E, H), jnp.float32),
                   jax.ShapeDtypeStruct((2, H), jnp.float32)],
    )(h_E, g2, gd,
      pe[0]["w"], _row2(pe[0]["b"]), pe[1]["w"], _row2(pe[1]["b"]),
      pe[2]["w"], _row2(pe[2]["b"]))

    h_E_out = pl.pallas_call(
        _f_body,
        grid=(GE,),
        in_specs=[
            _spec((BE, H), lambda i: (i, 0)),
            _spec((BE, H), lambda i: (i, 1)),
            wspec((2, H)),
            wspec((1, H)), wspec((1, H)),
        ],
        out_specs=_spec((BE, H), lambda i: (i, 0)),
        out_shape=jax.ShapeDtypeStruct((E, H), jnp.float32),
    )(x, g2, sumse, _row2(params["ebn_g"]), _row2(params["ebn_b"]))

    return (hv3, h_E_out)


# speedup vs baseline: 1.5421x; 1.1187x over previous
import functools

import jax
import jax.numpy as jnp
import numpy as np
from jax import lax
from jax.experimental import pallas as pl
from jax.experimental.pallas import tpu as pltpu
from jax.experimental.pallas import tpu_sc as plsc

N = 10000
E = 320000
G = 64
H = 128
HEADS = 4
DH = H // HEADS

ACC_W = 144
CHUNK = 128
NW = 32
BE = 2000
BN_ = 2000
GE = E // BE
GN = N // BN_
TILES = 16
NPAD = 10240
RPT = NPAD // TILES
ZR = 64


def _gelu(x):
    return 0.5 * x * (1.0 + lax.erf(x * np.float32(0.7071067811865476)))


def _mmb(a, b):
    return lax.dot_general(
        a.astype(jnp.bfloat16), b.astype(jnp.bfloat16),
        ((( 1,), (0,)), ((), ())),
        preferred_element_type=jnp.float32)


def _sc_gather(table, idx, rows, width, kc):
    R = kc * CHUNK
    U = rows // R
    pairs = (U + 2 * NW - 1) // (2 * NW)
    mesh = plsc.VectorSubcoreMesh(core_axis_name="c", subcore_axis_name="s")

    @functools.partial(
        pl.kernel,
        mesh=mesh,
        out_type=jax.ShapeDtypeStruct((rows, width), jnp.float32),
        scratch_types=[
            pltpu.VMEM((R,), jnp.int32),
            pltpu.VMEM((R,), jnp.int32),
            pltpu.VMEM((R, width), jnp.float32),
            pltpu.VMEM((R, width), jnp.float32),
            pltpu.SemaphoreType.DMA,
            pltpu.SemaphoreType.DMA,
            pltpu.SemaphoreType.DMA,
            pltpu.SemaphoreType.DMA,
            pltpu.SemaphoreType.DMA,
            pltpu.SemaphoreType.DMA,
        ],
    )
    def k(table_hbm, idx_hbm, out_hbm, i0, i1, b0, b1,
          is0, is1, gs0, gs1, os0, os1):
        wid = lax.axis_index("s") * 2 + lax.axis_index("c")
        ibufs = (i0, i1)
        bufs = (b0, b1)
        isems = (is0, is1)
        gsems = (gs0, gs1)
        osems = (os0, os1)

        def body(t, carry):
            for b in range(2):
                u = wid + (2 * t + b) * NW

                @pl.when(u < U)
                def _(u=u, b=b):
                    pltpu.async_copy(idx_hbm.at[pl.ds(u * R, R)],
                                     ibufs[b], isems[b])

            for b in range(2):
                u = wid + (2 * t + b) * NW

                @pl.when(u < U)
                def _(u=u, b=b):
                    @pl.when(u >= 2 * NW)
                    def _():
                        pltpu.make_async_copy(
                            bufs[b], out_hbm.at[pl.ds((u - 2 * NW) * R, R)],
                            osems[b]).wait()

                    pltpu.make_async_copy(idx_hbm.at[pl.ds(u * R, R)],
                                          ibufs[b], isems[b]).wait()
                    for kk in range(kc):
                        pltpu.async_copy(
                            table_hbm.at[ibufs[b].at[pl.ds(kk * CHUNK, CHUNK)]],
                            bufs[b].at[pl.ds(kk * CHUNK, CHUNK)], gsems[b])

            for b in range(2):
                u = wid + (2 * t + b) * NW

                @pl.when(u < U)
                def _(u=u, b=b):
                    for kk in range(kc):
                        pltpu.make_async_copy(
                            table_hbm.at[ibufs[b].at[pl.ds(kk * CHUNK, CHUNK)]],
                            bufs[b].at[pl.ds(kk * CHUNK, CHUNK)],
                            gsems[b]).wait()
                    pltpu.async_copy(bufs[b], out_hbm.at[pl.ds(u * R, R)],
                                     osems[b])

            return carry

        lax.fori_loop(0, pairs, body, 0)

        jm = (U - 1 - wid) // NW
        for b in range(2):
            jb = jnp.where((jm % 2) == b, jm, jm - 1)

            @pl.when(jb >= 0)
            def _(b=b, jb=jb):
                ul = wid + jb * NW
                pltpu.make_async_copy(
                    bufs[b], out_hbm.at[pl.ds(ul * R, R)], osems[b]).wait()

    return k(table, idx)


KSC = 1


def _sc_scatter128(vals, idx2d):
    R_ = KSC * CHUNK
    U = E // R_
    pairs = (((U + NW - 1) // NW) + 1) // 2
    mesh = plsc.VectorSubcoreMesh(core_axis_name="c", subcore_axis_name="s")

    @functools.partial(
        pl.kernel,
        mesh=mesh,
        out_type=jax.ShapeDtypeStruct((2, NPAD, H), jnp.float32),
        scratch_types=[
            pltpu.VMEM((KSC, CHUNK), jnp.int32),
            pltpu.VMEM((KSC, CHUNK), jnp.int32),
            pltpu.VMEM((R_, H), jnp.float32),
            pltpu.VMEM((R_, H), jnp.float32),
            pltpu.VMEM((ZR, H), jnp.float32),
            pltpu.VMEM_SHARED((NPAD, H), jnp.float32),
            pltpu.SemaphoreType.DMA,
            pltpu.SemaphoreType.DMA,
            pltpu.SemaphoreType.DMA,
            pltpu.SemaphoreType.DMA,
        ],
    )
    def k(vals_hbm, idx_hbm, out_hbm, i0, i1, v0, v1, zbuf, accum,
          is0, is1, vs0, vs1):
        c = lax.axis_index("c")
        s = lax.axis_index("s")
        wid = s * 2 + c

        def issue(u, ib, vb, isem, vsem):
            pltpu.async_copy(idx_hbm.at[pl.ds(u * KSC, KSC)], ib, isem)
            pltpu.async_copy(vals_hbm.at[pl.ds(u * R_, R_)], vb, vsem)

        def consume(u, ib, vb, isem, vsem):
            pltpu.make_async_copy(
                idx_hbm.at[pl.ds(u * KSC, KSC)], ib, isem).wait()
            pltpu.make_async_copy(
                vals_hbm.at[pl.ds(u * R_, R_)], vb, vsem).wait()
            for kk in range(KSC):
                pltpu.sync_copy(vb.at[pl.ds(kk * CHUNK, CHUNK)],
                                accum.at[ib.at[kk]], add=True)

        def zrow(i, carry):
            def zcol(j, cc):
                zbuf[i, pl.ds(j * 16, 16)] = jnp.zeros((16,), jnp.float32)
                return cc

            lax.fori_loop(0, H // 16, zcol, 0)
            return carry

        lax.fori_loop(0, ZR, zrow, 0)
        row0 = s * RPT
        for r in range(RPT // ZR):
            pltpu.sync_copy(zbuf, accum.at[pl.ds(row0 + r * ZR, ZR)])
        plsc.subcore_barrier()

        @pl.when(wid < U)
        def _():
            issue(wid, i0, v0, is0, vs0)

        def body(t, carry):
            ua = wid + (2 * t) * NW
            ub = wid + (2 * t + 1) * NW
            un = wid + (2 * t + 2) * NW

            @pl.when(ub < U)
            def _():
                issue(ub, i1, v1, is1, vs1)

            @pl.when(ua < U)
            def _():
                consume(ua, i0, v0, is0, vs0)

            @pl.when(un < U)
            def _():
                issue(un, i0, v0, is0, vs0)

            @pl.when(ub < U)
            def _():
                consume(ub, i1, v1, is1, vs1)

            return carry

        lax.fori_loop(0, pairs, body, 0)
        plsc.subcore_barrier()
        pltpu.sync_copy(accum.at[pl.ds(row0, RPT)],
                        out_hbm.at[c, pl.ds(row0, RPT)])

    return k(vals, idx2d)


def _e1_body(hE, hvs, hvd, w1, b1, w2, b2, w3, b3,
             v1, vb1, v2, vb2, v3, vb3, rep8, out, ew_out):
    he = hE[...]
    xs = hvs[...]
    xd = hvd[...]
    xw = jnp.concatenate([xs, he, xd], axis=1)
    a = _gelu(xw @ w1[...] + b1[...])
    a = _gelu(a @ w2[...] + b2[...])
    w8 = a @ w3[...] + b3[...]
    xv = jnp.concatenate([he, xd], axis=1)
    v = _gelu(xv @ v1[...] + vb1[...])
    v = _gelu(v @ v2[...] + vb2[...])
    v = v @ v3[...] + vb3[...]
    mask = (lax.broadcasted_iota(jnp.int32, (1, 8), 1) < HEADS)
    ew8 = jnp.exp(w8) * mask.astype(jnp.float32)
    out[...] = v * (ew8 @ rep8[...])
    ew_out[...] = jnp.concatenate(
        [ew8, jnp.zeros((ew8.shape[0], H - 8), jnp.float32)], axis=1)


def _a1_body(part, spart, hv, wo, rep4, t0_out, sums):
    i = pl.program_id(0)
    p = part[...]
    u = p[0] + p[1]
    q = spart[...]
    s4 = q[0, :, :HEADS] + q[1, :, :HEADS]
    agg = u / (s4 @ rep4[...] + 1e-9)
    t0 = hv[...] + agg @ wo[...]
    t0_out[...] = t0

    @pl.when(i == 0)
    def _():
        sums[...] = jnp.zeros_like(sums)

    sums[...] += jnp.concatenate(
        [jnp.sum(t0, axis=0, keepdims=True),
         jnp.sum(t0 * t0, axis=0, keepdims=True)], axis=0)


def _affine(sums, n, g, b):
    m = sums[0:1] / n
    v = sums[1:2] / n - m * m
    a = g / jnp.sqrt(v + 1e-5)
    return a, b - m * a


def _a2_body(t0, sums0, g0, b0, d0w, d0b, d1w, d1b, t2_out, sums):
    i = pl.program_id(0)
    a, bsh = _affine(sums0[...], float(N), g0[...], b0[...])
    t1 = a * t0[...] + bsh
    h = _gelu(t1 @ d0w[...] + d0b[...])
    t2 = t1 + h @ d1w[...] + d1b[...]
    t2_out[...] = t2

    @pl.when(i == 0)
    def _():
        sums[...] = jnp.zeros_like(sums)

    sums[...] += jnp.concatenate(
        [jnp.sum(t2, axis=0, keepdims=True),
         jnp.sum(t2 * t2, axis=0, keepdims=True)], axis=0)


def _b_body(t2, sums1, g1, b1, bid,
            vw1, vb1, vw2, vb2, vw3, vb3,
            ew1, eb1, ew2, eb2, ew3, eb3,
            hv2_out, hv3_out, tab_out):
    a, bsh = _affine(sums1[...], float(N), g1[...], b1[...])
    hv2 = a * t2[...] + bsh
    onehot = jnp.equal(
        bid[...], lax.broadcasted_iota(jnp.int32, (N, G), 1)
    ).astype(jnp.float32)
    dn = (((0,), (0,)), ((), ()))
    csum = lax.dot_general(onehot, hv2, dn)
    cnt = lax.dot_general(onehot, jnp.ones((N, 1), jnp.float32), dn)
    c = csum / jnp.maximum(cnt, 1.0)

    def mlp3(x, w1, bb1, w2, bb2, w3, bb3):
        x = _gelu(x @ w1[...] + bb1[...])
        x = _gelu(x @ w2[...] + bb2[...])
        return x @ w3[...] + bb3[...]

    svg = jax.nn.sigmoid(mlp3(c, vw1, vb1, vw2, vb2, vw3, vb3))
    c2 = c * svg
    seg = jax.nn.sigmoid(mlp3(c2, ew1, eb1, ew2, eb2, ew3, eb3))
    hv3 = hv2 * (onehot @ svg)
    hv2_out[...] = hv2
    hv3_out[...] = hv3
    tab_out[...] = jnp.concatenate([hv2, onehot @ seg], axis=1)


def _e2_body(hE, g2s, gd, w1, b1, w2, b2, w3, b3, x_out, sums):
    i = pl.program_id(0)
    he = hE[...]
    xin = jnp.concatenate([g2s[...], he, gd[...]], axis=1)
    a = _gelu(xin @ w1[...] + b1[...])
    a = _gelu(a @ w2[...] + b2[...])
    x = he + (a @ w3[...] + b3[...])
    x_out[...] = x

    @pl.when(i == 0)
    def _():
        sums[...] = jnp.zeros_like(sums)

    sums[...] += jnp.concatenate(
        [jnp.sum(x, axis=0, keepdims=True),
         jnp.sum(x * x, axis=0, keepdims=True)], axis=0)


def _f_body(x, gate, sumse, ge, be, out):
    a, bsh = _affine(sumse[...], float(E), ge[...], be[...])
    out[...] = (a * x[...] + bsh) * gate[...]


def _spec(shape, imap):
    return pl.BlockSpec(shape, imap)


def _row2(x):
    return x.reshape(1, -1)


def kernel(h_V, h_E, edge_idx, batch_id, params):
    src = edge_idx[0]
    dst = edge_idx[1]
    idx_all = edge_idx.reshape(2 * E)

    rep8 = np.zeros((8, H), np.float32)
    rep4 = np.zeros((HEADS, H), np.float32)
    for hh in range(HEADS):
        rep8[hh, hh * DH:(hh + 1) * DH] = 1.0
        rep4[hh, hh * DH:(hh + 1) * DH] = 1.0
    rep8 = jnp.asarray(rep8)
    rep4 = jnp.asarray(rep4)

    pb = params["att_Bias"]
    w3p = jnp.pad(pb[2]["w"], ((0, 0), (0, 8 - HEADS)))
    b3p = _row2(jnp.pad(pb[2]["b"], (0, 8 - HEADS)))
    pv = params["att_WV"]
    pe = params["edge"]
    d0, d1 = params["dense"]

    R1 = 3 * CHUNK
    rows1 = ((2 * E + R1 - 1) // R1) * R1
    idx1 = jnp.pad(idx_all, (0, rows1 - 2 * E))
    gath = _sc_gather(h_V, idx1, rows1, H, 3)

    wspec = lambda shp: _spec(shp, lambda i: (0, 0))
    p_acc, ew_acc = pl.pallas_call(
        _e1_body,
        grid=(GE,),
        in_specs=[
            _spec((BE, H), lambda i: (i, 0)),
            _spec((BE, H), lambda i: (i, 0)),
            _spec((BE, H), lambda i: (i + E // BE, 0)),
            wspec((3 * H, H)), wspec((1, H)),
            wspec((H, H)), wspec((1, H)),
            wspec((H, 8)), wspec((1, 8)),
            wspec((2 * H, H)), wspec((1, H)),
            wspec((H, H)), wspec((1, H)),
            wspec((H, H)), wspec((1, H)),
            wspec((8, H)),
        ],
        out_specs=[_spec((BE, H), lambda i: (i, 0)),
                   _spec((BE, H), lambda i: (i, 0))],
        out_shape=[jax.ShapeDtypeStruct((E, H), jnp.float32),
                   jax.ShapeDtypeStruct((E, H), jnp.float32)],
    )(h_E, gath, gath,
      pb[0]["w"], _row2(pb[0]["b"]), pb[1]["w"], _row2(pb[1]["b"]), w3p, b3p,
      pv[0]["w"], _row2(pv[0]["b"]), pv[1]["w"], _row2(pv[1]["b"]),
      pv[2]["w"], _row2(pv[2]["b"]), rep8)

    src2d = src.reshape(E // CHUNK, CHUNK)
    partials = _sc_scatter128(p_acc, src2d)
    spartials = _sc_scatter128(ew_acc, src2d)

    t0, sums0 = pl.pallas_call(
        _a1_body,
        grid=(GN,),
        in_specs=[
            _spec((2, BN_, H), lambda i: (0, i, 0)),
            _spec((2, BN_, H), lambda i: (0, i, 0)),
            _spec((BN_, H), lambda i: (i, 0)),
            wspec((H, H)),
            wspec((HEADS, H)),
        ],
        out_specs=[_spec((BN_, H), lambda i: (i, 0)),
                   _spec((2, H), lambda i: (0, 0))],
        out_shape=[jax.ShapeDtypeStruct((N, H), jnp.float32),
                   jax.ShapeDtypeStruct((2, H), jnp.float32)],
    )(partials, spartials, h_V, params["att_WO"], rep4)

    t2raw, sums1 = pl.pallas_call(
        _a2_body,
        grid=(GN,),
        in_specs=[
            _spec((BN_, H), lambda i: (i, 0)),
            wspec((2, H)),
            wspec((1, H)), wspec((1, H)),
            wspec((H, 4 * H)), wspec((1, 4 * H)),
            wspec((4 * H, H)), wspec((1, H)),
        ],
        out_specs=[_spec((BN_, H), lambda i: (i, 0)),
                   _spec((2, H), lambda i: (0, 0))],
        out_shape=[jax.ShapeDtypeStruct((N, H), jnp.float32),
                   jax.ShapeDtypeStruct((2, H), jnp.float32)],
    )(t0, sums0, _row2(params["bn0_g"]), _row2(params["bn0_b"]),
      d0["w"], _row2(d0["b"]), d1["w"], _row2(d1["b"]))

    pvg = params["vgate"]
    peg = params["egate"]
    hv2, hv3, tab = pl.pallas_call(
        _b_body,
        grid=(1,),
        in_specs=[
            _spec((N, H), lambda i: (0, 0)),
            wspec((2, H)),
            wspec((1, H)), wspec((1, H)),
            _spec((N, 1), lambda i: (0, 0)),
            wspec((H, H)), wspec((1, H)),
            wspec((H, H)), wspec((1, H)),
            wspec((H, H)), wspec((1, H)),
            wspec((H, H)), wspec((1, H)),
            wspec((H, H)), wspec((1, H)),
            wspec((H, H)), wspec((1, H)),
        ],
        out_specs=[_spec((N, H), lambda i: (0, 0)),
                   _spec((N, H), lambda i: (0, 0)),
                   _spec((N, 2 * H), lambda i: (0, 0))],
        out_shape=[jax.ShapeDtypeStruct((N, H), jnp.float32),
                   jax.ShapeDtypeStruct((N, H), jnp.float32),
                   jax.ShapeDtypeStruct((N, 2 * H), jnp.float32)],
    )(t2raw, sums1, _row2(params["bn1_g"]), _row2(params["bn1_b"]),
      batch_id.reshape(N, 1),
      pvg[0]["w"], _row2(pvg[0]["b"]), pvg[1]["w"], _row2(pvg[1]["b"]),
      pvg[2]["w"], _row2(pvg[2]["b"]),
      peg[0]["w"], _row2(peg[0]["b"]), peg[1]["w"], _row2(peg[1]["b"]),
      peg[2]["w"], _row2(peg[2]["b"]))

    g2 = _sc_gather(tab, src, E, 2 * H, 1)
    rows_d = ((E + R1 - 1) // R1) * R1
    gd = _sc_gather(hv2, jnp.pad(dst, (0, rows_d - E)), rows_d, H, 3)

    x, sumse = pl.pallas_call(
        _e2_body,
        grid=(GE,),
        in_specs=[
            _spec((BE, H), lambda i: (i, 0)),
            _spec((BE, H), lambda i: (i, 0)),
            _spec((BE, H), lambda i: (i, 0)),
            wspec((3 * H, H)), wspec((1, H)),
            wspec((H, H)), wspec((1, H)),
            wspec((H, H)), wspec((1, H)),
        ],
        out_specs=[_spec((BE, H), lambda i: (i, 0)),
                   _spec((2, H), lambda i: (0, 0))],
        out_shape=[jax.ShapeDtypeStruct((E, H), jnp.float32),
                   jax.ShapeDtypeStruct((2, H), jnp.float32)],
    )(h_E, g2, gd,
      pe[0]["w"], _row2(pe[0]["b"]), pe[1]["w"], _row2(pe[1]["b"]),
      pe[2]["w"], _row2(pe[2]["b"]))

    h_E_out = pl.pallas_call(
        _f_body,
        grid=(GE,),
        in_specs=[
            _spec((BE, H), lambda i: (i, 0)),
            _spec((BE, H), lambda i: (i, 1)),
            wspec((2, H)),
            wspec((1, H)), wspec((1, H)),
        ],
        out_specs=_spec((BE, H), lambda i: (i, 0)),
        out_shape=jax.ShapeDtypeStruct((E, H), jnp.float32),
    )(x, g2, sumse, _row2(params["ebn_g"]), _row2(params["ebn_b"]))

    return (hv3, h_E_out)
